# Initial kernel scaffold; baseline (speedup 1.0000x reference)
import functools

import jax
import jax.numpy as jnp
from jax import lax
from jax.experimental import pallas as pl
from jax.experimental.pallas import tpu as pltpu
from jax.experimental.pallas import tpu_sc as plsc

B = 128
V = 100000
CHUNK = 50000
NVEC = CHUNK // 16
L = 16
NW = 32
ROWS_PER_W = B // NW


def _build_sc_kernel():
  mesh = plsc.VectorSubcoreMesh(core_axis_name="c", subcore_axis_name="s")

  @functools.partial(
      pl.kernel,
      mesh=mesh,
      out_type=jax.ShapeDtypeStruct((2, L), jnp.float32),
      scratch_types=[
          pltpu.VMEM((B,), jnp.int32),
          pltpu.VMEM((8,), jnp.float32),
          pltpu.VMEM((2, CHUNK), jnp.float32),
          pltpu.VMEM((L,), jnp.float32),
          pltpu.VMEM((L, L), jnp.float32),
          pltpu.VMEM_SHARED((L, L), jnp.float32),
          pltpu.SemaphoreType.DMA,
          pltpu.SemaphoreType.DMA,
      ],
  )
  def sc_kernel(outputs_hbm, targets_hbm, out_hbm,
                tgt_v, vbuf, cbuf, part_v, red_v, shared, sem0, sem1):
    c = lax.axis_index("c")
    s = lax.axis_index("s")
    w = s * 2 + c

    pltpu.sync_copy(targets_hbm, tgt_v)
    iota = lax.iota(jnp.int32, L)
    thr = ((iota == 0).astype(jnp.int32)
           + (iota == 1).astype(jnp.int32) * 5
           + (iota == 2).astype(jnp.int32) * 10)

    acc3 = jnp.zeros((L,), jnp.float32)
    for i in range(ROWS_PER_W):
      r = w * ROWS_PER_W + i
      tvec = plsc.load_gather(tgt_v, [jnp.full((L,), r, jnp.int32)])
      t = jnp.max(tvec)
      vbase = (t // 8) * 8
      pltpu.sync_copy(outputs_hbm.at[r, pl.ds(vbase, 8)], vbuf)
      v_splat = plsc.load_gather(vbuf, [jnp.full((L,), t - vbase, jnp.int32)])

      cp0 = pltpu.async_copy(outputs_hbm.at[r, pl.ds(0, CHUNK)],
                             cbuf.at[0], sem0)
      cp1 = pltpu.async_copy(outputs_hbm.at[r, pl.ds(CHUNK, CHUNK)],
                             cbuf.at[1], sem1)
      rank = jnp.int32(0)
      for kk, cp in ((0, cp0), (1, cp1)):
        cp.wait()
        buf = cbuf.at[kk]
        tb = t - kk * CHUNK
        cb = jnp.clip(tb, 0, CHUNK)
        nA = cb // 16
        in_mid = jnp.logical_and(tb >= 0, tb < CHUNK)
        startB = nA + in_mid.astype(jnp.int32)

        def body_ge(i2, a, buf=buf):
          x = buf[pl.ds(i2 * 16, 16)]
          return a + (x >= v_splat).astype(jnp.int32)

        def body_gt(i2, a, buf=buf):
          x = buf[pl.ds(i2 * 16, 16)]
          return a + (x > v_splat).astype(jnp.int32)

        accv = lax.fori_loop(0, nA, body_ge, jnp.zeros((L,), jnp.int32))
        bsafe = jnp.minimum(nA, NVEC - 1)
        xb = buf[pl.ds(bsafe * 16, 16)]
        rb = tb - bsafe * 16
        mb = (((xb >= v_splat) & (iota < rb))
              | ((xb > v_splat) & (iota > rb)))
        accv = accv + jnp.where(in_mid, mb.astype(jnp.int32),
                                jnp.zeros((L,), jnp.int32))
        accv = lax.fori_loop(startB, NVEC, body_gt, accv)
        rank = rank + jnp.sum(accv)

      rk = jnp.full((L,), rank, jnp.int32)
      acc3 = acc3 + (rk < thr).astype(jnp.float32)

    part_v[...] = acc3
    pltpu.sync_copy(part_v, shared.at[s])
    plsc.subcore_barrier()

    @pl.when(s == 0)
    def _():
      pltpu.sync_copy(shared, red_v)
      tot = jnp.zeros((L,), jnp.float32)
      for i in range(L):
        tot = tot + red_v[i, :]
      part_v[...] = tot
      pltpu.sync_copy(part_v, out_hbm.at[c])

  return sc_kernel


@jax.jit
def kernel(outputs, targets):
  sc = _build_sc_kernel()
  res = sc(outputs, targets.astype(jnp.int32))
  tot = res[0] + res[1]
  return (tot[0], tot[1], tot[2])


# speedup vs baseline: 1.9105x; 1.9105x over previous
import functools

import jax
import jax.numpy as jnp
from jax import lax
from jax.experimental import pallas as pl
from jax.experimental.pallas import tpu as pltpu
from jax.experimental.pallas import tpu_sc as plsc

B = 128
V = 100000
CHUNK = 50000
NVEC = CHUNK // 16
L = 16
NW = 32
ROWS_PER_W = B // NW
NCHUNKS = ROWS_PER_W * 2
UNROLL = 8


def _build_sc_kernel():
  mesh = plsc.VectorSubcoreMesh(core_axis_name="c", subcore_axis_name="s")

  @functools.partial(
      pl.kernel,
      mesh=mesh,
      compiler_params=pltpu.CompilerParams(needs_layout_passes=False),
      out_type=jax.ShapeDtypeStruct((2 * L,), jnp.float32),
      scratch_types=[
          pltpu.VMEM((B,), jnp.int32),
          pltpu.VMEM((ROWS_PER_W * L,), jnp.float32),
          pltpu.VMEM((CHUNK,), jnp.float32),
          pltpu.VMEM((CHUNK,), jnp.float32),
          pltpu.VMEM((L,), jnp.float32),
          pltpu.VMEM((L * L,), jnp.float32),
          pltpu.VMEM_SHARED((L * L,), jnp.float32),
          pltpu.SemaphoreType.DMA,
          pltpu.SemaphoreType.DMA,
      ],
  )
  def sc_kernel(outputs_hbm, targets_hbm, out_hbm,
                tgt_v, vbuf, cbuf0, cbuf1, part_v, red_v, shared,
                sem0, sem1):
    c = lax.axis_index("c")
    s = lax.axis_index("s")
    w = s * 2 + c

    pltpu.sync_copy(targets_hbm, tgt_v)
    iota = lax.iota(jnp.int32, L)
    thr = ((iota == 0).astype(jnp.int32)
           + (iota == 1).astype(jnp.int32) * 5
           + (iota == 2).astype(jnp.int32) * 10)

    ts = []
    vbases = []
    for i in range(ROWS_PER_W):
      r = w * ROWS_PER_W + i
      tb0 = jnp.minimum(r, B - L)
      tvec = tgt_v[pl.ds(tb0, L)]
      t = jnp.max(jnp.where(iota == r - tb0, tvec, -1))
      vbase = (t // 8) * 8
      pltpu.sync_copy(outputs_hbm.at[pl.ds(r * V + vbase, 8)],
                      vbuf.at[pl.ds(i * L, 8)])
      ts.append(t)
      vbases.append(vbase)
    vsplats = []
    for i in range(ROWS_PER_W):
      vvec = vbuf[pl.ds(i * L, L)]
      v = jnp.max(jnp.where(iota == ts[i] - vbases[i], vvec, -jnp.inf))
      vsplats.append(jnp.full((L,), v))

    bufs = (cbuf0, cbuf1)
    sems = (sem0, sem1)

    def chunk_src(j):
      row = w * ROWS_PER_W + j // 2
      return outputs_hbm.at[pl.ds(row * V + (j % 2) * CHUNK, CHUNK)]

    cps = [pltpu.async_copy(chunk_src(0), cbuf0, sem0),
           pltpu.async_copy(chunk_src(1), cbuf1, sem1)]

    ranks = [jnp.int32(0)] * ROWS_PER_W
    for j in range(NCHUNKS):
      cps[j % 2].wait()
      buf = bufs[j % 2]
      i = j // 2
      t = ts[i]
      v_splat = vsplats[i]
      tb = t - (j % 2) * CHUNK
      cb = jnp.clip(tb, 0, CHUNK)
      nA = cb // 16
      in_mid = jnp.logical_and(tb >= 0, tb < CHUNK)
      startB = nA + in_mid.astype(jnp.int32)

      def body_ge(i2, a, buf=buf, v_splat=v_splat):
        x = buf[pl.ds(i2 * 16, 16)]
        return a + (x >= v_splat).astype(jnp.int32)

      def body_gt(i2, a, buf=buf, v_splat=v_splat):
        x = buf[pl.ds(i2 * 16, 16)]
        return a + (x > v_splat).astype(jnp.int32)

      accv = plsc.parallel_loop(0, nA, unroll=UNROLL,
                                carry=jnp.zeros((L,), jnp.int32))(body_ge)
      bsafe = jnp.minimum(nA, NVEC - 1)
      xb = buf[pl.ds(bsafe * 16, 16)]
      rb = tb - bsafe * 16
      in_mid_v = jnp.full((L,), in_mid)
      mlo = jnp.logical_and(xb >= v_splat, iota < rb)
      mhi = jnp.logical_and(xb > v_splat, iota > rb)
      mb = jnp.logical_and(jnp.logical_or(mlo, mhi), in_mid_v)
      accv = accv + mb.astype(jnp.int32)
      accv = plsc.parallel_loop(startB, NVEC, unroll=UNROLL,
                                carry=accv)(body_gt)
      ranks[i] = ranks[i] + jnp.sum(accv)

      if j + 2 < NCHUNKS:
        cps[j % 2] = pltpu.async_copy(chunk_src(j + 2), buf, sems[j % 2])

    acc3 = jnp.zeros((L,), jnp.float32)
    for i in range(ROWS_PER_W):
      rk = jnp.full((L,), ranks[i], jnp.int32)
      acc3 = acc3 + (rk < thr).astype(jnp.float32)

    part_v[...] = acc3
    pltpu.sync_copy(part_v, shared.at[pl.ds(s * L, L)])
    plsc.subcore_barrier()

    @pl.when(s == 0)
    def _():
      pltpu.sync_copy(shared, red_v)
      tot = jnp.zeros((L,), jnp.float32)
      for i in range(L):
        tot = tot + red_v[pl.ds(i * L, L)]
      part_v[...] = tot
      pltpu.sync_copy(part_v, out_hbm.at[pl.ds(c * L, L)])

  return sc_kernel


@jax.jit
def kernel(outputs, targets):
  sc = _build_sc_kernel()
  res = sc(outputs.reshape(-1), targets.astype(jnp.int32))
  tot = res[:L] + res[L:]
  return (tot[0], tot[1], tot[2])


# speedup vs baseline: 2.9536x; 1.5460x over previous
import functools

import jax
import jax.numpy as jnp
from jax import lax
from jax.experimental import pallas as pl
from jax.experimental.pallas import tpu as pltpu
from jax.experimental.pallas import tpu_sc as plsc

B = 128
V = 100000
L = 16
HALF = 49920
CW = 4096
NCW = HALF // CW
REM = HALF - NCW * CW
TAIL0 = 2 * HALF
UNROLL = 8
NEG_INF = float("-inf")


def _build_sc_kernel():
  mesh = plsc.VectorSubcoreMesh(core_axis_name="c", subcore_axis_name="s")

  @functools.partial(
      pl.kernel,
      mesh=mesh,
      compiler_params=pltpu.CompilerParams(needs_layout_passes=False),
      out_type=jax.ShapeDtypeStruct((2 * L,), jnp.float32),
      scratch_types=[
          pltpu.VMEM((B,), jnp.int32),
          pltpu.VMEM((8, 128), jnp.float32),
          pltpu.VMEM((8, CW), jnp.float32),
          pltpu.VMEM((8, CW), jnp.float32),
          pltpu.VMEM((8, 128), jnp.float32),
          pltpu.VMEM((8, 32), jnp.float32),
          pltpu.VMEM((L,), jnp.float32),
          pltpu.VMEM((L * L,), jnp.float32),
          pltpu.VMEM_SHARED((L * L,), jnp.float32),
          pltpu.SemaphoreType.DMA,
          pltpu.SemaphoreType.DMA,
      ],
  )
  def sc_kernel(outputs_hbm, targets_hbm, out_hbm,
                tgt_v, vblk, cbuf0, cbuf1, tailA, tailB, part_v, red_v,
                shared, sem0, sem1):
    c = lax.axis_index("c")
    s = lax.axis_index("s")
    rb = s // 2 + 8 * c
    p = s % 2
    r0 = rb * 8
    col0 = p * HALF

    pltpu.sync_copy(targets_hbm, tgt_v)
    iota = lax.iota(jnp.int32, L)

    tbase = jnp.minimum(r0, B - L)
    d = r0 - tbase
    ts_raw = tgt_v[pl.ds(tbase, L)]

    ts_vec = jnp.zeros((L,), jnp.int32)
    vs_vec = jnp.zeros((L,), jnp.float32)
    for j in range(8):
      t_j = jnp.max(jnp.where(iota == j + d, ts_raw, -1))
      tile_c = (t_j // 128) * 128
      pltpu.sync_copy(outputs_hbm.at[pl.ds(r0, 8), pl.ds(tile_c, 128)], vblk)
      within = t_j - tile_c
      seg = (within // 16) * 16
      vvec = vblk[j, pl.ds(seg, L)]
      v_j = jnp.max(jnp.where(iota == within - seg, vvec, NEG_INF))
      ts_vec = ts_vec + jnp.where(iota == j, t_j, 0)
      vs_vec = vs_vec + jnp.where(iota == j, v_j, jnp.float32(0))

    def count_rows(buf, cstart, cw, ranks):
      nv = cw // 16

      def row_body(j, rv):
        t = jnp.max(jnp.where(iota == j, ts_vec, -1))
        v = jnp.max(jnp.where(iota == j, vs_vec, NEG_INF))
        v_splat = jnp.full((L,), v)
        tb = t - cstart
        cb = jnp.clip(tb, 0, cw)
        nA = cb // 16
        in_mid = jnp.logical_and(tb >= 0, tb < cw)
        startB = nA + in_mid.astype(jnp.int32)

        def body_ge(i2, a):
          x = buf[j, pl.ds(i2 * 16, 16)]
          return a + (x >= v_splat).astype(jnp.int32)

        def body_gt(i2, a):
          x = buf[j, pl.ds(i2 * 16, 16)]
          return a + (x > v_splat).astype(jnp.int32)

        accv = plsc.parallel_loop(0, nA, unroll=UNROLL,
                                  carry=jnp.zeros((L,), jnp.int32))(body_ge)
        bsafe = jnp.minimum(nA, nv - 1)
        xb = buf[j, pl.ds(bsafe * 16, 16)]
        rbnd = tb - bsafe * 16
        in_mid_v = jnp.full((L,), in_mid)
        mlo = jnp.logical_and(xb >= v_splat, iota < rbnd)
        mhi = jnp.logical_and(xb > v_splat, iota > rbnd)
        mb = jnp.logical_and(jnp.logical_or(mlo, mhi), in_mid_v)
        accv = accv + mb.astype(jnp.int32)
        accv = plsc.parallel_loop(startB, nv, unroll=UNROLL,
                                  carry=accv)(body_gt)
        rank = jnp.sum(accv)
        return rv + jnp.where(iota == j, rank, 0)

      return lax.fori_loop(0, 8, row_body, ranks)

    widths = [CW] * NCW + [REM]
    offs = [col0 + k * CW for k in range(NCW)] + [col0 + NCW * CW]
    bufs = (cbuf0, cbuf1)
    sems = (sem0, sem1)

    def fire(k):
      return pltpu.async_copy(
          outputs_hbm.at[pl.ds(r0, 8), pl.ds(offs[k], widths[k])],
          bufs[k % 2].at[:, pl.ds(0, widths[k])], sems[k % 2])

    cps = [fire(0), fire(1)]
    ranks = jnp.zeros((L,), jnp.int32)
    for k in range(len(widths)):
      cps[k % 2].wait()
      ranks = count_rows(bufs[k % 2], offs[k], widths[k], ranks)
      if k + 2 < len(widths):
        cps[k % 2] = fire(k + 2)

    @pl.when(p == 1)
    def _():
      pltpu.sync_copy(outputs_hbm.at[pl.ds(r0, 8), pl.ds(TAIL0, 128)], tailA)
      pltpu.sync_copy(outputs_hbm.at[pl.ds(r0, 8), pl.ds(TAIL0 + 128, 32)],
                      tailB)
      rt = count_rows(tailA, jnp.int32(TAIL0), 128, jnp.zeros((L,), jnp.int32))
      rt = count_rows(tailB, jnp.int32(TAIL0 + 128), 32, rt)
      part_v[...] = (ranks + rt).astype(jnp.float32)

    @pl.when(p == 0)
    def _():
      part_v[...] = ranks.astype(jnp.float32)

    pltpu.sync_copy(part_v, shared.at[pl.ds(s * L, L)])
    plsc.subcore_barrier()

    @pl.when(s == 0)
    def _():
      pltpu.sync_copy(shared, red_v)
      lane8 = iota < 8
      acc1 = jnp.zeros((L,), jnp.float32)
      acc5 = jnp.zeros((L,), jnp.float32)
      acc10 = jnp.zeros((L,), jnp.float32)
      one = jnp.full((L,), jnp.float32(1))
      zero = jnp.zeros((L,), jnp.float32)
      for k in range(8):
        rv = red_v[pl.ds((2 * k) * L, L)] + red_v[pl.ds((2 * k + 1) * L, L)]
        m1 = jnp.logical_and(rv < 1.0, lane8)
        m5 = jnp.logical_and(rv < 5.0, lane8)
        m10 = jnp.logical_and(rv < 10.0, lane8)
        acc1 = acc1 + jnp.where(m1, one, zero)
        acc5 = acc5 + jnp.where(m5, one, zero)
        acc10 = acc10 + jnp.where(m10, one, zero)
      s1 = jnp.sum(acc1)
      s5 = jnp.sum(acc5)
      s10 = jnp.sum(acc10)
      tot = (jnp.where(iota == 0, s1, jnp.float32(0))
             + jnp.where(iota == 1, s5, jnp.float32(0))
             + jnp.where(iota == 2, s10, jnp.float32(0)))
      part_v[...] = tot
      pltpu.sync_copy(part_v, out_hbm.at[pl.ds(c * L, L)])

  return sc_kernel


@jax.jit
def kernel(outputs, targets):
  sc = _build_sc_kernel()
  res = sc(outputs, targets.astype(jnp.int32))
  tot = res[:L] + res[L:]
  return (tot[0], tot[1], tot[2])


# speedup vs baseline: 3.0468x; 1.0315x over previous
import functools

import jax
import jax.numpy as jnp
from jax import lax
from jax.experimental import pallas as pl
from jax.experimental.pallas import tpu as pltpu
from jax.experimental.pallas import tpu_sc as plsc

B = 128
V = 100000
L = 16
HALF = 49920
CW = 4096
NCW = HALF // CW
REM = HALF - NCW * CW
TAIL0 = 2 * HALF
UNROLL = 8
NEG_INF = float("-inf")


def _build_sc_kernel():
  mesh = plsc.VectorSubcoreMesh(core_axis_name="c", subcore_axis_name="s")

  @functools.partial(
      pl.kernel,
      mesh=mesh,
      compiler_params=pltpu.CompilerParams(needs_layout_passes=False,
                                           skip_device_barrier=True),
      out_type=jax.ShapeDtypeStruct((2 * L,), jnp.float32),
      scratch_types=[
          pltpu.VMEM((B,), jnp.int32),
          pltpu.VMEM((8, 8 * 128), jnp.float32),
          pltpu.VMEM((8, CW), jnp.float32),
          pltpu.VMEM((8, CW), jnp.float32),
          pltpu.VMEM((8, CW), jnp.float32),
          pltpu.VMEM((8, 128), jnp.float32),
          pltpu.VMEM((8, 32), jnp.float32),
          pltpu.VMEM((L,), jnp.float32),
          pltpu.VMEM((L * L,), jnp.float32),
          pltpu.VMEM_SHARED((L * L,), jnp.float32),
          pltpu.SemaphoreType.DMA,
          pltpu.SemaphoreType.DMA,
          pltpu.SemaphoreType.DMA,
          pltpu.SemaphoreType.DMA,
      ],
  )
  def sc_kernel(outputs_hbm, targets_hbm, out_hbm,
                tgt_v, vblk, cbuf0, cbuf1, cbuf2, tailA, tailB, part_v, red_v,
                shared, sem0, sem1, sem2, semv):
    c = lax.axis_index("c")
    s = lax.axis_index("s")
    rb = s // 2 + 8 * c
    p = s % 2
    r0 = rb * 8
    col0 = p * HALF

    pltpu.sync_copy(targets_hbm, tgt_v)
    iota = lax.iota(jnp.int32, L)

    tbase = jnp.minimum(r0, B - L)
    d = r0 - tbase
    ts_raw = tgt_v[pl.ds(tbase, L)]

    ts_vec = jnp.zeros((L,), jnp.int32)
    vs_vec = jnp.zeros((L,), jnp.float32)
    tjs = []
    vcps = []
    for j in range(8):
      t_j = jnp.max(jnp.where(iota == j + d, ts_raw, -1))
      tile_c = (t_j // 128) * 128
      vcps.append(pltpu.async_copy(
          outputs_hbm.at[pl.ds(r0, 8), pl.ds(tile_c, 128)],
          vblk.at[:, pl.ds(j * 128, 128)], semv))
      tjs.append(t_j)
    for j in range(8):
      vcps[j].wait()
    for j in range(8):
      t_j = tjs[j]
      within = t_j - (t_j // 128) * 128
      seg = (within // 16) * 16
      vvec = vblk[j, pl.ds(j * 128 + seg, L)]
      v_j = jnp.max(jnp.where(iota == within - seg, vvec, NEG_INF))
      ts_vec = ts_vec + jnp.where(iota == j, t_j, 0)
      vs_vec = vs_vec + jnp.where(iota == j, v_j, jnp.float32(0))

    def count_rows(buf, cstart, cw, ranks):
      nv = cw // 16

      def row_body(j, rv):
        t = jnp.max(jnp.where(iota == j, ts_vec, -1))
        v = jnp.max(jnp.where(iota == j, vs_vec, NEG_INF))
        v_splat = jnp.full((L,), v)
        tb = t - cstart
        cb = jnp.clip(tb, 0, cw)
        nA = cb // 16
        in_mid = jnp.logical_and(tb >= 0, tb < cw)
        startB = nA + in_mid.astype(jnp.int32)

        def body_ge(i2, a):
          x = buf[j, pl.ds(i2 * 16, 16)]
          return a + (x >= v_splat).astype(jnp.int32)

        def body_gt(i2, a):
          x = buf[j, pl.ds(i2 * 16, 16)]
          return a + (x > v_splat).astype(jnp.int32)

        accv = plsc.parallel_loop(0, nA, unroll=UNROLL,
                                  carry=jnp.zeros((L,), jnp.int32))(body_ge)
        bsafe = jnp.minimum(nA, nv - 1)
        xb = buf[j, pl.ds(bsafe * 16, 16)]
        rbnd = tb - bsafe * 16
        in_mid_v = jnp.full((L,), in_mid)
        mlo = jnp.logical_and(xb >= v_splat, iota < rbnd)
        mhi = jnp.logical_and(xb > v_splat, iota > rbnd)
        mb = jnp.logical_and(jnp.logical_or(mlo, mhi), in_mid_v)
        accv = accv + mb.astype(jnp.int32)
        accv = plsc.parallel_loop(startB, nv, unroll=UNROLL,
                                  carry=accv)(body_gt)
        rank = jnp.sum(accv)
        return rv + jnp.where(iota == j, rank, 0)

      return lax.fori_loop(0, 8, row_body, ranks)

    widths = [CW] * NCW + [REM]
    offs = [col0 + k * CW for k in range(NCW)] + [col0 + NCW * CW]
    bufs = (cbuf0, cbuf1, cbuf2)
    sems = (sem0, sem1, sem2)
    NBUF = 3

    def fire(k):
      return pltpu.async_copy(
          outputs_hbm.at[pl.ds(r0, 8), pl.ds(offs[k], widths[k])],
          bufs[k % NBUF].at[:, pl.ds(0, widths[k])], sems[k % NBUF])

    cps = [fire(k) for k in range(NBUF)]
    ranks = jnp.zeros((L,), jnp.int32)
    for k in range(len(widths)):
      cps[k % NBUF].wait()
      ranks = count_rows(bufs[k % NBUF], offs[k], widths[k], ranks)
      if k + NBUF < len(widths):
        cps[k % NBUF] = fire(k + NBUF)

    @pl.when(p == 1)
    def _():
      pltpu.sync_copy(outputs_hbm.at[pl.ds(r0, 8), pl.ds(TAIL0, 128)], tailA)
      pltpu.sync_copy(outputs_hbm.at[pl.ds(r0, 8), pl.ds(TAIL0 + 128, 32)],
                      tailB)
      rt = count_rows(tailA, jnp.int32(TAIL0), 128, jnp.zeros((L,), jnp.int32))
      rt = count_rows(tailB, jnp.int32(TAIL0 + 128), 32, rt)
      part_v[...] = (ranks + rt).astype(jnp.float32)

    @pl.when(p == 0)
    def _():
      part_v[...] = ranks.astype(jnp.float32)

    pltpu.sync_copy(part_v, shared.at[pl.ds(s * L, L)])
    plsc.subcore_barrier()

    @pl.when(s == 0)
    def _():
      pltpu.sync_copy(shared, red_v)
      lane8 = iota < 8
      acc1 = jnp.zeros((L,), jnp.float32)
      acc5 = jnp.zeros((L,), jnp.float32)
      acc10 = jnp.zeros((L,), jnp.float32)
      one = jnp.full((L,), jnp.float32(1))
      zero = jnp.zeros((L,), jnp.float32)
      for k in range(8):
        rv = red_v[pl.ds((2 * k) * L, L)] + red_v[pl.ds((2 * k + 1) * L, L)]
        m1 = jnp.logical_and(rv < 1.0, lane8)
        m5 = jnp.logical_and(rv < 5.0, lane8)
        m10 = jnp.logical_and(rv < 10.0, lane8)
        acc1 = acc1 + jnp.where(m1, one, zero)
        acc5 = acc5 + jnp.where(m5, one, zero)
        acc10 = acc10 + jnp.where(m10, one, zero)
      s1 = jnp.sum(acc1)
      s5 = jnp.sum(acc5)
      s10 = jnp.sum(acc10)
      tot = (jnp.where(iota == 0, s1, jnp.float32(0))
             + jnp.where(iota == 1, s5, jnp.float32(0))
             + jnp.where(iota == 2, s10, jnp.float32(0)))
      part_v[...] = tot
      pltpu.sync_copy(part_v, out_hbm.at[pl.ds(c * L, L)])

  return sc_kernel


@jax.jit
def kernel(outputs, targets):
  sc = _build_sc_kernel()
  res = sc(outputs, targets.astype(jnp.int32))
  tot = res[:L] + res[L:]
  return (tot[0], tot[1], tot[2])
